# Initial kernel scaffold; baseline (speedup 1.0000x reference)
import jax
import jax.numpy as jnp
from jax.experimental import pallas as pl


def kernel(x, edge_index, W1, b1, Wl, bl, Wr):
    raise NotImplementedError("write your pallas kernel here")


# speedup vs baseline: 19.3921x; 19.3921x over previous
import functools

import jax
import jax.numpy as jnp
from jax import lax
from jax.experimental import pallas as pl
from jax.experimental.pallas import tpu as pltpu
from jax.experimental.pallas import tpu_sc as plsc

_NC = 2
_NS = 16
_NW = _NC * _NS
_CHUNK = 128
_F = 16


def _mesh():
    return plsc.VectorSubcoreMesh(core_axis_name="c", subcore_axis_name="s")


def _sc_count(dst, n_pad):
    E = dst.shape[0]
    nchunks = E // _CHUNK
    assert nchunks * _CHUNK == E
    trips = nchunks // _NW
    extra = nchunks % _NW
    rpt = n_pad // _NS

    @functools.partial(
        pl.kernel,
        mesh=_mesh(),
        out_type=jax.ShapeDtypeStruct((_NC, n_pad, _F), jnp.float32),
        compiler_params=pltpu.CompilerParams(use_tc_tiling_on_sc=False),
        scratch_types=[
            pltpu.VMEM((_CHUNK,), jnp.int32),
            pltpu.VMEM((_CHUNK, _F), jnp.float32),
            pltpu.VMEM((rpt, _F), jnp.float32),
            pltpu.VMEM_SHARED((n_pad, _F), jnp.float32),
        ],
    )
    def k(dst_hbm, out_hbm, idx_v, ones_v, stage_v, acc_sh):
        cid = lax.axis_index("c")
        sid = lax.axis_index("s")
        wid = sid * _NC + cid

        def fill_ones(i, c):
            ones_v[i] = jnp.ones((_F,), jnp.float32)
            return c

        lax.fori_loop(0, _CHUNK, fill_ones, 0)

        def fill_zero(i, c):
            stage_v[i] = jnp.zeros((_F,), jnp.float32)
            return c

        lax.fori_loop(0, rpt, fill_zero, 0)
        pltpu.sync_copy(stage_v, acc_sh.at[pl.ds(sid * rpt, rpt)])
        plsc.subcore_barrier()

        def step(j, c):
            base = pl.multiple_of((wid + _NW * j) * _CHUNK, _CHUNK)
            pltpu.sync_copy(dst_hbm.at[pl.ds(base, _CHUNK)], idx_v)
            pltpu.sync_copy(ones_v, acc_sh.at[idx_v], add=True)
            return c

        lax.fori_loop(0, trips, step, 0)
        if extra:
            @pl.when(wid < extra)
            def _():
                base = pl.multiple_of((trips * _NW + wid) * _CHUNK, _CHUNK)
                pltpu.sync_copy(dst_hbm.at[pl.ds(base, _CHUNK)], idx_v)
                pltpu.sync_copy(ones_v, acc_sh.at[idx_v], add=True)

        plsc.subcore_barrier()
        pltpu.sync_copy(acc_sh.at[pl.ds(sid * rpt, rpt)], stage_v)
        pltpu.sync_copy(stage_v, out_hbm.at[cid, pl.ds(sid * rpt, rpt)])

    return k(dst)


def _sc_segsum(table, src, dst, n_pad):
    E = src.shape[0]
    nchunks = E // _CHUNK
    trips = nchunks // _NW
    extra = nchunks % _NW
    rpt = n_pad // _NS

    @functools.partial(
        pl.kernel,
        mesh=_mesh(),
        out_type=jax.ShapeDtypeStruct((_NC, n_pad, _F), jnp.float32),
        compiler_params=pltpu.CompilerParams(use_tc_tiling_on_sc=False),
        scratch_types=[
            pltpu.VMEM((_CHUNK,), jnp.int32),
            pltpu.VMEM((_CHUNK,), jnp.int32),
            pltpu.VMEM((_CHUNK, _F), jnp.float32),
            pltpu.VMEM((rpt, _F), jnp.float32),
            pltpu.VMEM_SHARED((n_pad, _F), jnp.float32),
            pltpu.SemaphoreType.DMA,
        ],
    )
    def k(table_hbm, src_hbm, dst_hbm, out_hbm,
          sidx_v, didx_v, rows_v, stage_v, acc_sh, sem):
        cid = lax.axis_index("c")
        sid = lax.axis_index("s")
        wid = sid * _NC + cid

        def fill_zero(i, c):
            stage_v[i] = jnp.zeros((_F,), jnp.float32)
            return c

        lax.fori_loop(0, rpt, fill_zero, 0)
        pltpu.sync_copy(stage_v, acc_sh.at[pl.ds(sid * rpt, rpt)])
        plsc.subcore_barrier()

        def do_chunk(base):
            pltpu.sync_copy(src_hbm.at[pl.ds(base, _CHUNK)], sidx_v)
            pltpu.sync_copy(dst_hbm.at[pl.ds(base, _CHUNK)], didx_v)
            pltpu.async_copy(table_hbm.at[sidx_v], rows_v, sem).wait()
            pltpu.sync_copy(rows_v, acc_sh.at[didx_v], add=True)

        def step(j, c):
            do_chunk(pl.multiple_of((wid + _NW * j) * _CHUNK, _CHUNK))
            return c

        lax.fori_loop(0, trips, step, 0)
        if extra:
            @pl.when(wid < extra)
            def _():
                do_chunk(pl.multiple_of((trips * _NW + wid) * _CHUNK, _CHUNK))

        plsc.subcore_barrier()
        pltpu.sync_copy(acc_sh.at[pl.ds(sid * rpt, rpt)], stage_v)
        pltpu.sync_copy(stage_v, out_hbm.at[cid, pl.ds(sid * rpt, rpt)])

    return k(table, src, dst)


def _tc_mm(x, W1, n_pad):
    N = x.shape[0]

    def body(x_ref, w_ref, hx_ref):
        hx = jnp.dot(x_ref[...], w_ref[...], preferred_element_type=jnp.float32)
        hx_ref[0:N, :] = hx
        hx_ref[N:n_pad, :] = jnp.zeros((n_pad - N, _F), jnp.float32)

    return pl.pallas_call(
        body,
        out_shape=jax.ShapeDtypeStruct((n_pad, _F), jnp.float32),
    )(x, W1)


def _tc_prep(hx, cnt_p):
    n_pad = hx.shape[0]

    def body(hx_ref, cnt_ref, hxs_ref, dinvb_ref, invcb_ref):
        cnt = cnt_ref[0] + cnt_ref[1]
        dinvb = lax.rsqrt(cnt + 1.0)
        dinvb_ref[...] = dinvb
        invcb_ref[...] = 1.0 / jnp.maximum(cnt, 1.0)
        hxs_ref[...] = hx_ref[...] * dinvb

    shp = jax.ShapeDtypeStruct((n_pad, _F), jnp.float32)
    return pl.pallas_call(body, out_shape=(shp, shp, shp))(hx, cnt_p)


def _tc_comb(t1_p, hx, dinvb, b1):
    n_pad = hx.shape[0]

    def body(t1_ref, hx_ref, dinvb_ref, b1_ref, h_ref):
        d = dinvb_ref[...]
        t1 = t1_ref[0] + t1_ref[1]
        h_ref[...] = d * t1 + d * d * hx_ref[...] + b1_ref[...][None, :]

    return pl.pallas_call(
        body,
        out_shape=jax.ShapeDtypeStruct((n_pad, _F), jnp.float32),
    )(t1_p, hx, dinvb, b1)


def _tc_out(t2_p, h, invcb, Wl, bl, Wr):
    n_pad = h.shape[0]
    C = Wl.shape[1]

    def body(t2_ref, h_ref, invcb_ref, wl_ref, bl_ref, wr_ref, o_ref):
        mean = (t2_ref[0] + t2_ref[1]) * invcb_ref[...]
        h = h_ref[...]
        o = (jnp.dot(mean, wl_ref[...], preferred_element_type=jnp.float32)
             + jnp.dot(h, wr_ref[...], preferred_element_type=jnp.float32)
             + bl_ref[...][None, :])
        m = jnp.max(o, axis=1, keepdims=True)
        lse = m + jnp.log(jnp.sum(jnp.exp(o - m), axis=1, keepdims=True))
        o_ref[...] = o - lse

    return pl.pallas_call(
        body,
        out_shape=jax.ShapeDtypeStruct((n_pad, C), jnp.float32),
    )(t2_p, h, invcb, Wl, bl, Wr)


def kernel(x, edge_index, W1, b1, Wl, bl, Wr):
    N = x.shape[0]
    n_pad = ((N + 255) // 256) * 256
    src = edge_index[0]
    dst = edge_index[1]

    hx = _tc_mm(x, W1, n_pad)
    cnt_p = _sc_count(dst, n_pad)
    hxs, dinvb, invcb = _tc_prep(hx, cnt_p)
    t1_p = _sc_segsum(hxs, src, dst, n_pad)
    h = _tc_comb(t1_p, hx, dinvb, b1)
    t2_p = _sc_segsum(h, src, dst, n_pad)
    out = _tc_out(t2_p, h, invcb, Wl, bl, Wr)
    return out[:N]


# speedup vs baseline: 48.0736x; 2.4790x over previous
import functools

import jax
import jax.numpy as jnp
from jax import lax
from jax.experimental import pallas as pl
from jax.experimental.pallas import tpu as pltpu
from jax.experimental.pallas import tpu_sc as plsc

_NC = 2
_NS = 16
_NW = _NC * _NS
_CHUNK = 128
_F = 16


def _mesh():
    return plsc.VectorSubcoreMesh(core_axis_name="c", subcore_axis_name="s")


_TRIPS = 80


def _zero_acc(stage_v, acc_sh, sid, rpt):
    def fill_zero(i, c):
        stage_v[i] = jnp.zeros((_F,), jnp.float32)
        return c

    lax.fori_loop(0, rpt, fill_zero, 0)
    pltpu.sync_copy(stage_v, acc_sh.at[pl.ds(sid * rpt, rpt)])
    plsc.subcore_barrier()


def _copy_out(stage_v, acc_sh, out_hbm, cid, sid, rpt):
    plsc.subcore_barrier()
    pltpu.sync_copy(acc_sh.at[pl.ds(sid * rpt, rpt)], stage_v)
    pltpu.sync_copy(stage_v, out_hbm.at[cid, pl.ds(sid * rpt, rpt)])


def _sc_count(dst3, n_pad):
    rpt = n_pad // _NS

    @functools.partial(
        pl.kernel,
        mesh=_mesh(),
        out_type=jax.ShapeDtypeStruct((_NC, n_pad, _F), jnp.float32),
        compiler_params=pltpu.CompilerParams(use_tc_tiling_on_sc=False),
        scratch_types=[
            pltpu.VMEM((_TRIPS, _CHUNK), jnp.int32),
            pltpu.VMEM((_CHUNK, _F), jnp.float32),
            pltpu.VMEM((rpt, _F), jnp.float32),
            pltpu.VMEM_SHARED((n_pad, _F), jnp.float32),
            pltpu.SemaphoreType.DMA,
            pltpu.SemaphoreType.DMA,
            pltpu.SemaphoreType.DMA,
            pltpu.SemaphoreType.DMA,
        ],
    )
    def k(dst_hbm, out_hbm, didx_v, ones_v, stage_v, acc_sh, s0, s1, s2, s3):
        cid = lax.axis_index("c")
        sid = lax.axis_index("s")
        wid = sid * _NC + cid
        sems = [s0, s1, s2, s3]

        def fill_ones(i, c):
            ones_v[i] = jnp.ones((_F,), jnp.float32)
            return c

        lax.fori_loop(0, _CHUNK, fill_ones, 0)
        pltpu.sync_copy(dst_hbm.at[wid, :, :], didx_v)
        _zero_acc(stage_v, acc_sh, sid, rpt)

        def swait(sem):
            pltpu.make_async_copy(ones_v, acc_sh.at[didx_v.at[0]], sem).wait()

        def outer(g4, c):
            g = g4 * 4
            for b in range(4):
                j = g + b

                @pl.when(j >= 4)
                def _():
                    swait(sems[b])

                pltpu.async_copy(ones_v, acc_sh.at[didx_v.at[j]], sems[b],
                                 add=True)
            return c

        lax.fori_loop(0, _TRIPS // 4, outer, 0)
        for b in range(4):
            swait(sems[b])
        _copy_out(stage_v, acc_sh, out_hbm, cid, sid, rpt)

    return k(dst3)


def _sc_segsum(table, src3, dst3, n_pad):
    rpt = n_pad // _NS

    @functools.partial(
        pl.kernel,
        mesh=_mesh(),
        out_type=jax.ShapeDtypeStruct((_NC, n_pad, _F), jnp.float32),
        compiler_params=pltpu.CompilerParams(use_tc_tiling_on_sc=False),
        scratch_types=[
            pltpu.VMEM((_TRIPS, _CHUNK), jnp.int32),
            pltpu.VMEM((_TRIPS, _CHUNK), jnp.int32),
            pltpu.VMEM((_CHUNK, _F), jnp.float32),
            pltpu.VMEM((_CHUNK, _F), jnp.float32),
            pltpu.VMEM((_CHUNK, _F), jnp.float32),
            pltpu.VMEM((_CHUNK, _F), jnp.float32),
            pltpu.VMEM((rpt, _F), jnp.float32),
            pltpu.VMEM_SHARED((n_pad, _F), jnp.float32),
            pltpu.SemaphoreType.DMA,
            pltpu.SemaphoreType.DMA,
            pltpu.SemaphoreType.DMA,
            pltpu.SemaphoreType.DMA,
            pltpu.SemaphoreType.DMA,
            pltpu.SemaphoreType.DMA,
            pltpu.SemaphoreType.DMA,
            pltpu.SemaphoreType.DMA,
        ],
    )
    def k(table_hbm, src_hbm, dst_hbm, out_hbm,
          sidx_v, didx_v, r0, r1, r2, r3, stage_v, acc_sh,
          g0, g1, g2, g3, s0, s1, s2, s3):
        cid = lax.axis_index("c")
        sid = lax.axis_index("s")
        wid = sid * _NC + cid
        rows = [r0, r1, r2, r3]
        gsem = [g0, g1, g2, g3]
        ssem = [s0, s1, s2, s3]

        pltpu.sync_copy(src_hbm.at[wid, :, :], sidx_v)
        pltpu.sync_copy(dst_hbm.at[wid, :, :], didx_v)
        _zero_acc(stage_v, acc_sh, sid, rpt)

        def gwait(b):
            pltpu.make_async_copy(table_hbm.at[sidx_v.at[0]], rows[b],
                                  gsem[b]).wait()

        def swait(b):
            pltpu.make_async_copy(rows[b], acc_sh.at[didx_v.at[0]],
                                  ssem[b]).wait()

        pltpu.async_copy(table_hbm.at[sidx_v.at[0]], rows[0], gsem[0])
        pltpu.async_copy(table_hbm.at[sidx_v.at[1]], rows[1], gsem[1])

        def outer(g4, c):
            g = g4 * 4
            for b in range(4):
                j = g + b
                bf = (b + 2) % 4

                @pl.when(j + 2 < _TRIPS)
                def _():
                    @pl.when(j >= 2)
                    def _():
                        swait(bf)

                    pltpu.async_copy(table_hbm.at[sidx_v.at[j + 2]],
                                     rows[bf], gsem[bf])

                gwait(b)
                pltpu.async_copy(rows[b], acc_sh.at[didx_v.at[j]], ssem[b],
                                 add=True)
            return c

        lax.fori_loop(0, _TRIPS // 4, outer, 0)
        for b in range(4):
            swait(b)
        _copy_out(stage_v, acc_sh, out_hbm, cid, sid, rpt)

    return k(table, src3, dst3)


def _tc_mm(x, W1, n_pad):
    N = x.shape[0]

    def body(x_ref, w_ref, hx_ref):
        hx = jnp.dot(x_ref[...], w_ref[...], preferred_element_type=jnp.float32)
        hx_ref[0:N, :] = hx
        hx_ref[N:n_pad, :] = jnp.zeros((n_pad - N, _F), jnp.float32)

    return pl.pallas_call(
        body,
        out_shape=jax.ShapeDtypeStruct((n_pad, _F), jnp.float32),
    )(x, W1)


def _tc_prep(hx, cnt_p):
    n_pad = hx.shape[0]

    def body(hx_ref, cnt_ref, hxs_ref, dinvb_ref, invcb_ref):
        cnt = cnt_ref[0] + cnt_ref[1]
        dinvb = lax.rsqrt(cnt + 1.0)
        dinvb_ref[...] = dinvb
        invcb_ref[...] = 1.0 / jnp.maximum(cnt, 1.0)
        hxs_ref[...] = hx_ref[...] * dinvb

    shp = jax.ShapeDtypeStruct((n_pad, _F), jnp.float32)
    return pl.pallas_call(body, out_shape=(shp, shp, shp))(hx, cnt_p)


def _tc_comb(t1_p, hx, dinvb, b1):
    n_pad = hx.shape[0]

    def body(t1_ref, hx_ref, dinvb_ref, b1_ref, h_ref):
        d = dinvb_ref[...]
        t1 = t1_ref[0] + t1_ref[1]
        h_ref[...] = d * t1 + d * d * hx_ref[...] + b1_ref[...][None, :]

    return pl.pallas_call(
        body,
        out_shape=jax.ShapeDtypeStruct((n_pad, _F), jnp.float32),
    )(t1_p, hx, dinvb, b1)


def _tc_out(t2_p, h, invcb, Wl, bl, Wr):
    n_pad = h.shape[0]
    C = Wl.shape[1]

    def body(t2_ref, h_ref, invcb_ref, wl_ref, bl_ref, wr_ref, o_ref):
        mean = (t2_ref[0] + t2_ref[1]) * invcb_ref[...]
        h = h_ref[...]
        o = (jnp.dot(mean, wl_ref[...], preferred_element_type=jnp.float32)
             + jnp.dot(h, wr_ref[...], preferred_element_type=jnp.float32)
             + bl_ref[...][None, :])
        m = jnp.max(o, axis=1, keepdims=True)
        lse = m + jnp.log(jnp.sum(jnp.exp(o - m), axis=1, keepdims=True))
        o_ref[...] = o - lse

    return pl.pallas_call(
        body,
        out_shape=jax.ShapeDtypeStruct((n_pad, C), jnp.float32),
    )(t2_p, h, invcb, Wl, bl, Wr)


def kernel(x, edge_index, W1, b1, Wl, bl, Wr):
    N = x.shape[0]
    E = edge_index.shape[1]
    n_pad = ((N + 255) // 256) * 256
    e_pad = _NW * _TRIPS * _CHUNK
    pad = e_pad - E
    assert 0 <= pad
    idx = jnp.arange(pad, dtype=jnp.int32)
    src3 = jnp.concatenate([edge_index[0], idx % N]).reshape(_NW, _TRIPS, _CHUNK)
    dst3 = jnp.concatenate(
        [edge_index[1], N + idx % (n_pad - N)]).reshape(_NW, _TRIPS, _CHUNK)

    hx = _tc_mm(x, W1, n_pad)
    cnt_p = _sc_count(dst3, n_pad)
    hxs, dinvb, invcb = _tc_prep(hx, cnt_p)
    t1_p = _sc_segsum(hxs, src3, dst3, n_pad)
    h = _tc_comb(t1_p, hx, dinvb, b1)
    t2_p = _sc_segsum(h, src3, dst3, n_pad)
    out = _tc_out(t2_p, h, invcb, Wl, bl, Wr)
    return out[:N]
